# Initial kernel scaffold; baseline (speedup 1.0000x reference)
import jax
import jax.numpy as jnp
from jax.experimental import pallas as pl


def kernel(features, edges, W1, b1, W2, b2, W3, b3):
    raise NotImplementedError("write your pallas kernel here")


# speedup vs baseline: 14.0711x; 14.0711x over previous
import functools

import jax
import jax.numpy as jnp
from jax import lax
from jax.experimental import pallas as pl
from jax.experimental.pallas import tpu as pltpu
from jax.experimental.pallas import tpu_sc as plsc

N = 10000
E = 320000
D = 128

NP = 10240
ROWS_PER_TILE = NP // 16
NW = 32
CHUNK = 128
NCHUNK = 79
EW = NCHUNK * CHUNK
EPAD = NW * EW

BLK = 256
GRID = NP // BLK

_mesh = plsc.VectorSubcoreMesh(core_axis_name="c", subcore_axis_name="s")


def _tile_slices(sid):
    return pl.ds(sid * ROWS_PER_TILE, ROWS_PER_TILE)


def _sc_deg_body(dst_hbm, zeros1_hbm, deg_out, dst_v, ones_v, acc_sh):
    cid = lax.axis_index("c")
    sid = lax.axis_index("s")
    wid = cid * 16 + sid
    sl = _tile_slices(sid)
    pltpu.sync_copy(zeros1_hbm.at[sl], acc_sh.at[sl])
    for i in range(CHUNK // 16):
        ones_v[pl.ds(i * 16, 16)] = jnp.ones((16,), jnp.float32)
    pltpu.sync_copy(dst_hbm.at[wid], dst_v)
    plsc.subcore_barrier()

    @pl.loop(0, NCHUNK)
    def _(j):
        pltpu.sync_copy(ones_v, acc_sh.at[dst_v.at[j]], add=True)

    plsc.subcore_barrier()
    pltpu.sync_copy(acc_sh.at[sl], deg_out.at[cid, sl])


def _sc_deg(dst_p, zeros1):
    return pl.kernel(
        _sc_deg_body,
        out_type=jax.ShapeDtypeStruct((2, NP), jnp.float32),
        mesh=_mesh,
        scratch_types=[
            pltpu.VMEM((NCHUNK, CHUNK), jnp.int32),
            pltpu.VMEM((CHUNK,), jnp.float32),
            pltpu.VMEM_SHARED((NP,), jnp.float32),
        ],
    )(dst_p, zeros1)


def _sc_agg_body(u_hbm, src_hbm, dst_hbm, zeros_hbm, part_out,
                 src_v, dst_v, rows_v, sem, acc_sh):
    cid = lax.axis_index("c")
    sid = lax.axis_index("s")
    wid = cid * 16 + sid
    sl = _tile_slices(sid)
    pltpu.sync_copy(zeros_hbm.at[sl], acc_sh.at[sl])
    pltpu.sync_copy(src_hbm.at[wid], src_v)
    pltpu.sync_copy(dst_hbm.at[wid], dst_v)
    plsc.subcore_barrier()

    @pl.loop(0, NCHUNK)
    def _(j):
        pltpu.async_copy(u_hbm.at[src_v.at[j]], rows_v, sem).wait()
        pltpu.sync_copy(rows_v, acc_sh.at[dst_v.at[j]], add=True)

    plsc.subcore_barrier()
    pltpu.sync_copy(acc_sh.at[sl], part_out.at[cid, sl])


def _sc_agg(u, src_p, dst_p, zeros):
    return pl.kernel(
        _sc_agg_body,
        out_type=jax.ShapeDtypeStruct((2, NP, D), jnp.float32),
        mesh=_mesh,
        scratch_types=[
            pltpu.VMEM((NCHUNK, CHUNK), jnp.int32),
            pltpu.VMEM((NCHUNK, CHUNK), jnp.int32),
            pltpu.VMEM((CHUNK, D), jnp.float32),
            pltpu.SemaphoreType.DMA,
            pltpu.VMEM_SHARED((NP, D), jnp.float32),
        ],
    )(u, src_p, dst_p, zeros)


def _sc_zagg_body(z_hbm, src_hbm, dst_hbm, zeros1_hbm, q_out,
                  src_v, dst_v, vals_v, sem, acc_sh):
    cid = lax.axis_index("c")
    sid = lax.axis_index("s")
    wid = cid * 16 + sid
    sl = _tile_slices(sid)
    pltpu.sync_copy(zeros1_hbm.at[sl], acc_sh.at[sl])
    pltpu.sync_copy(src_hbm.at[wid], src_v)
    pltpu.sync_copy(dst_hbm.at[wid], dst_v)
    plsc.subcore_barrier()

    @pl.loop(0, NCHUNK)
    def _(j):
        pltpu.async_copy(z_hbm.at[src_v.at[j]], vals_v, sem).wait()
        pltpu.sync_copy(vals_v, acc_sh.at[dst_v.at[j]], add=True)

    plsc.subcore_barrier()
    pltpu.sync_copy(acc_sh.at[sl], q_out.at[cid, sl])


def _sc_zagg(z, src_p, dst_p, zeros1):
    return pl.kernel(
        _sc_zagg_body,
        out_type=jax.ShapeDtypeStruct((2, NP), jnp.float32),
        mesh=_mesh,
        scratch_types=[
            pltpu.VMEM((NCHUNK, CHUNK), jnp.int32),
            pltpu.VMEM((NCHUNK, CHUNK), jnp.int32),
            pltpu.VMEM((CHUNK,), jnp.float32),
            pltpu.SemaphoreType.DMA,
            pltpu.VMEM_SHARED((NP,), jnp.float32),
        ],
    )(z, src_p, dst_p, zeros1)


def _t1_body(x_ref, w1_ref, degp_ref, dinv_ref, u1_ref):
    deg = degp_ref[...][0] + degp_ref[...][1] + 1.0
    dinv = lax.rsqrt(deg)[:, None]
    xw = jnp.dot(x_ref[...], w1_ref[...], preferred_element_type=jnp.float32)
    dinv_ref[...] = dinv
    u1_ref[...] = xw * dinv


def _t1(x, W1, degp):
    return pl.pallas_call(
        _t1_body,
        grid=(GRID,),
        in_specs=[
            pl.BlockSpec((BLK, D), lambda i: (i, 0)),
            pl.BlockSpec((D, D), lambda i: (0, 0)),
            pl.BlockSpec((2, BLK), lambda i: (0, i)),
        ],
        out_specs=[
            pl.BlockSpec((BLK, 1), lambda i: (i, 0)),
            pl.BlockSpec((BLK, D), lambda i: (i, 0)),
        ],
        out_shape=[
            jax.ShapeDtypeStruct((NP, 1), jnp.float32),
            jax.ShapeDtypeStruct((NP, D), jnp.float32),
        ],
    )(x, W1, degp)


def _t2_body(p_ref, u_ref, dinv_ref, b_ref, w_ref, out_ref):
    p = p_ref[...]
    dinv = dinv_ref[...]
    h = jnp.maximum(dinv * (p[0] + p[1] + u_ref[...]) + b_ref[...], 0.0)
    out_ref[...] = dinv * jnp.dot(h, w_ref[...], preferred_element_type=jnp.float32)


def _t2(part, u, dinv, b, W):
    return pl.pallas_call(
        _t2_body,
        grid=(GRID,),
        in_specs=[
            pl.BlockSpec((2, BLK, D), lambda i: (0, i, 0)),
            pl.BlockSpec((BLK, D), lambda i: (i, 0)),
            pl.BlockSpec((BLK, 1), lambda i: (i, 0)),
            pl.BlockSpec((1, D), lambda i: (0, 0)),
            pl.BlockSpec((D, D), lambda i: (0, 0)),
        ],
        out_specs=pl.BlockSpec((BLK, D), lambda i: (i, 0)),
        out_shape=jax.ShapeDtypeStruct((NP, D), jnp.float32),
    )(part, u, dinv, b, W)


def _t3_body(p_ref, u_ref, dinv_ref, b_ref, w3_ref, z_ref):
    p = p_ref[...]
    dinv = dinv_ref[...]
    h = jnp.maximum(dinv * (p[0] + p[1] + u_ref[...]) + b_ref[...], 0.0)
    z_ref[...] = dinv * jnp.sum(h * w3_ref[...], axis=1, keepdims=True)


def _t3(part, u, dinv, b, w3row):
    return pl.pallas_call(
        _t3_body,
        grid=(GRID,),
        in_specs=[
            pl.BlockSpec((2, BLK, D), lambda i: (0, i, 0)),
            pl.BlockSpec((BLK, D), lambda i: (i, 0)),
            pl.BlockSpec((BLK, 1), lambda i: (i, 0)),
            pl.BlockSpec((1, D), lambda i: (0, 0)),
            pl.BlockSpec((1, D), lambda i: (0, 0)),
        ],
        out_specs=pl.BlockSpec((BLK, 1), lambda i: (i, 0)),
        out_shape=jax.ShapeDtypeStruct((NP, 1), jnp.float32),
    )(part, u, dinv, b, w3row)


def _t4_body(q_ref, z_ref, dinv_ref, b3_ref, out_ref):
    q = q_ref[...]
    qs = (q[0] + q[1])[:, None]
    out_ref[...] = jnp.maximum(
        dinv_ref[...] * (qs + z_ref[...]) + b3_ref[0, 0], 0.0)


def _t4(qp, z, dinv, b3):
    return pl.pallas_call(
        _t4_body,
        grid=(GRID,),
        in_specs=[
            pl.BlockSpec((2, BLK), lambda i: (0, i)),
            pl.BlockSpec((BLK, 1), lambda i: (i, 0)),
            pl.BlockSpec((BLK, 1), lambda i: (i, 0)),
            pl.BlockSpec((1, 1), lambda i: (0, 0)),
        ],
        out_specs=pl.BlockSpec((BLK, 1), lambda i: (i, 0)),
        out_shape=jax.ShapeDtypeStruct((NP, 1), jnp.float32),
    )(qp, z, dinv, b3)


def kernel(features, edges, W1, b1, W2, b2, W3, b3):
    x = jnp.pad(features, ((0, NP - N), (0, 0)))
    src = edges[0].astype(jnp.int32)
    dst = edges[1].astype(jnp.int32)
    pad = EPAD - E
    src_p = jnp.concatenate([src, jnp.zeros((pad,), jnp.int32)]).reshape(NW, NCHUNK, CHUNK)
    dst_p = jnp.concatenate([dst, jnp.full((pad,), N, jnp.int32)]).reshape(NW, NCHUNK, CHUNK)
    zeros = jnp.zeros((NP, D), jnp.float32)
    zeros1 = jnp.zeros((NP,), jnp.float32)

    degp = _sc_deg(dst_p, zeros1)
    dinv, u1 = _t1(x, W1, degp)
    p1 = _sc_agg(u1, src_p, dst_p, zeros)
    u2 = _t2(p1, u1, dinv, b1.reshape(1, D), W2)
    p2 = _sc_agg(u2, src_p, dst_p, zeros)
    z3 = _t3(p2, u2, dinv, b2.reshape(1, D), W3.reshape(1, D))
    qp = _sc_zagg(z3.reshape(NP), src_p, dst_p, zeros1)
    out = _t4(qp, z3, dinv, b3.reshape(1, 1))
    return out[:N]
